# Initial kernel scaffold; baseline (speedup 1.0000x reference)
import jax
import jax.numpy as jnp
from jax.experimental import pallas as pl


def kernel(x0, x1, y):
    raise NotImplementedError("write your pallas kernel here")


# speedup vs baseline: 1.3315x; 1.3315x over previous
import functools

import jax
import jax.numpy as jnp
from jax import lax
from jax.experimental import pallas as pl
from jax.experimental.pallas import tpu as pltpu

_LANES = 128
_SUBLANES = 8
_BLOCK_ROWS = 2048


def _loss_block_kernel(
    x0_ref, x1_ref, y_ref, out_ref, *, margin, n_rows, block_rows, mask_tail
):
    i = pl.program_id(0)

    x0 = x0_ref[...].astype(jnp.float32)
    x1 = x1_ref[...].astype(jnp.float32)
    diff = x0 - x1
    sq = diff * diff

    ones = jnp.ones((sq.shape[1], 1), jnp.float32)
    dist_sq = jnp.dot(sq, ones, preferred_element_type=jnp.float32)
    dist = jnp.sqrt(dist_sq)
    clamped = jnp.maximum(margin - dist, 0.0)
    hinge = clamped * clamped

    if mask_tail:
        row_ids = lax.broadcasted_iota(jnp.int32, dist_sq.shape, 0) + i * block_rows
        valid = row_ids < n_rows
        dist_sq = jnp.where(valid, dist_sq, 0.0)
        hinge = jnp.where(valid, hinge, 0.0)

    y_row = y_ref[...].reshape(1, block_rows)
    weighted = jnp.dot(
        y_row, dist_sq - hinge, preferred_element_type=jnp.float32
    )
    partial = weighted + jnp.sum(hinge, keepdims=True)

    out_ref[...] = jnp.broadcast_to(partial, out_ref.shape)


def _round_up(v, m):
    return ((v + m - 1) // m) * m


def kernel(x0, x1, y):
    n, d = x0.shape
    margin = 1.0

    tn = min(_BLOCK_ROWS, _round_up(n, _SUBLANES))
    grid_n = pl.cdiv(n, tn)
    mask_tail = (grid_n * tn) != n

    y_flat = y.reshape(-1).astype(jnp.float32)
    if grid_n * tn != n:
        y_flat = jnp.pad(y_flat, (0, grid_n * tn - n))
    y3 = y_flat.reshape(grid_n, 1, tn)

    body = functools.partial(
        _loss_block_kernel,
        margin=float(margin),
        n_rows=int(n),
        block_rows=int(tn),
        mask_tail=bool(mask_tail),
    )

    partials = pl.pallas_call(
        body,
        out_shape=jax.ShapeDtypeStruct((grid_n * _SUBLANES, _LANES), jnp.float32),
        grid=(grid_n,),
        in_specs=[
            pl.BlockSpec((tn, d), lambda i: (i, 0)),
            pl.BlockSpec((tn, d), lambda i: (i, 0)),
            pl.BlockSpec((1, 1, tn), lambda i: (i, 0, 0)),
        ],
        out_specs=pl.BlockSpec((_SUBLANES, _LANES), lambda i: (i, 0)),
        compiler_params=pltpu.CompilerParams(
            dimension_semantics=("parallel",),
        ),
    )(x0, x1, y3)

    per_block = partials.reshape(grid_n, _SUBLANES, _LANES)[:, 0, 0]
    return jnp.sum(per_block) * (0.5 / n)


# speedup vs baseline: 1.3332x; 1.0013x over previous
import functools

import jax
import jax.numpy as jnp
from jax import lax
from jax.experimental import pallas as pl
from jax.experimental.pallas import tpu as pltpu

_LANES = 128
_SUBLANES = 8
_BLOCK_ROWS = 2048


def _loss_block_kernel(
    x0_ref, x1_ref, y_ref, out_ref, *, margin, n_rows, block_rows, mask_tail
):
    i = pl.program_id(0)

    x0 = x0_ref[...].astype(jnp.float32)
    x1 = x1_ref[...].astype(jnp.float32)
    diff = x0 - x1
    sq = diff * diff

    ones = jnp.ones((sq.shape[1], 1), jnp.float32)
    dist_sq = jnp.dot(sq, ones, preferred_element_type=jnp.float32)
    dist = jnp.sqrt(dist_sq)
    clamped = jnp.maximum(margin - dist, 0.0)
    hinge = clamped * clamped

    if mask_tail:
        row_ids = lax.broadcasted_iota(jnp.int32, dist_sq.shape, 0) + i * block_rows
        valid = row_ids < n_rows
        dist_sq = jnp.where(valid, dist_sq, 0.0)
        hinge = jnp.where(valid, hinge, 0.0)

    y_row = y_ref[...].astype(jnp.float32).reshape(1, block_rows)
    weighted = jnp.dot(
        y_row, dist_sq - hinge, preferred_element_type=jnp.float32
    )
    partial = weighted + jnp.sum(hinge, keepdims=True)

    out_ref[...] = jnp.broadcast_to(partial, out_ref.shape)


def _round_up(v, m):
    return ((v + m - 1) // m) * m


def kernel(x0, x1, y):
    n, d = x0.shape
    margin = 1.0

    tn = min(_BLOCK_ROWS, _round_up(n, _SUBLANES))
    grid_n = pl.cdiv(n, tn)
    mask_tail = (grid_n * tn) != n

    y_flat = y.reshape(-1)
    if grid_n * tn != n:
        y_flat = jnp.pad(y_flat.astype(jnp.float32), (0, grid_n * tn - n))

    body = functools.partial(
        _loss_block_kernel,
        margin=float(margin),
        n_rows=int(n),
        block_rows=int(tn),
        mask_tail=bool(mask_tail),
    )

    partials = pl.pallas_call(
        body,
        out_shape=jax.ShapeDtypeStruct((grid_n * _SUBLANES, _LANES), jnp.float32),
        grid=(grid_n,),
        in_specs=[
            pl.BlockSpec((tn, d), lambda i: (i, 0)),
            pl.BlockSpec((tn, d), lambda i: (i, 0)),
            pl.BlockSpec((tn,), lambda i: (i,)),
        ],
        out_specs=pl.BlockSpec((_SUBLANES, _LANES), lambda i: (i, 0)),
        compiler_params=pltpu.CompilerParams(
            dimension_semantics=("parallel",),
        ),
    )(x0, x1, y_flat)

    per_block = partials.reshape(grid_n, _SUBLANES, _LANES)[:, 0, 0]
    return jnp.sum(per_block) * (0.5 / n)


# speedup vs baseline: 1.5729x; 1.1798x over previous
import functools

import jax
import jax.numpy as jnp
from jax import lax
from jax.experimental import pallas as pl
from jax.experimental.pallas import tpu as pltpu

_LANES = 128
_SUBLANES = 8
_BLOCK_ROWS = 4096


def _loss_block_kernel(
    x0_ref, x1_ref, y_ref, out_ref, *, margin, n_rows, block_rows, mask_tail
):
    i = pl.program_id(0)

    x0 = x0_ref[...].astype(jnp.float32)
    x1 = x1_ref[...].astype(jnp.float32)
    diff = x0 - x1
    sq = diff * diff

    ones = jnp.ones((sq.shape[1], 1), jnp.float32)
    dist_sq = jnp.dot(sq, ones, preferred_element_type=jnp.float32)
    dist = jnp.sqrt(dist_sq)
    clamped = jnp.maximum(margin - dist, 0.0)
    hinge = clamped * clamped

    if mask_tail:
        row_ids = lax.broadcasted_iota(jnp.int32, dist_sq.shape, 0) + i * block_rows
        valid = row_ids < n_rows
        dist_sq = jnp.where(valid, dist_sq, 0.0)
        hinge = jnp.where(valid, hinge, 0.0)

    y_row = y_ref[...].astype(jnp.float32).reshape(1, block_rows)
    weighted = jnp.dot(
        y_row, dist_sq - hinge, preferred_element_type=jnp.float32
    )
    partial = weighted + jnp.sum(hinge, keepdims=True)

    out_ref[...] = jnp.broadcast_to(partial, out_ref.shape)


def _round_up(v, m):
    return ((v + m - 1) // m) * m


def kernel(x0, x1, y):
    n, d = x0.shape
    margin = 1.0

    tn = min(_BLOCK_ROWS, _round_up(n, _SUBLANES))
    grid_n = pl.cdiv(n, tn)
    mask_tail = (grid_n * tn) != n

    y_flat = y.reshape(-1)
    if grid_n * tn != n:
        y_flat = jnp.pad(y_flat.astype(jnp.float32), (0, grid_n * tn - n))

    body = functools.partial(
        _loss_block_kernel,
        margin=float(margin),
        n_rows=int(n),
        block_rows=int(tn),
        mask_tail=bool(mask_tail),
    )

    partials = pl.pallas_call(
        body,
        out_shape=jax.ShapeDtypeStruct((grid_n * _SUBLANES, _LANES), jnp.float32),
        grid=(grid_n,),
        in_specs=[
            pl.BlockSpec((tn, d), lambda i: (i, 0)),
            pl.BlockSpec((tn, d), lambda i: (i, 0)),
            pl.BlockSpec((tn,), lambda i: (i,)),
        ],
        out_specs=pl.BlockSpec((_SUBLANES, _LANES), lambda i: (i, 0)),
        compiler_params=pltpu.CompilerParams(
            dimension_semantics=("parallel",),
        ),
    )(x0, x1, y_flat)

    per_block = partials.reshape(grid_n, _SUBLANES, _LANES)[:, 0, 0]
    return jnp.sum(per_block) * (0.5 / n)


# speedup vs baseline: 1.5782x; 1.0034x over previous
import functools

import jax
import jax.numpy as jnp
from jax import lax
from jax.experimental import pallas as pl
from jax.experimental.pallas import tpu as pltpu

_LANES = 128
_SUBLANES = 8
_BLOCK_ROWS = 8192


def _loss_block_kernel(
    x0_ref, x1_ref, y_ref, out_ref, *, margin, n_rows, block_rows, mask_tail
):
    i = pl.program_id(0)

    x0 = x0_ref[...].astype(jnp.float32)
    x1 = x1_ref[...].astype(jnp.float32)
    diff = x0 - x1
    sq = diff * diff

    ones = jnp.ones((sq.shape[1], 1), jnp.float32)
    dist_sq = jnp.dot(sq, ones, preferred_element_type=jnp.float32)
    dist = jnp.sqrt(dist_sq)
    clamped = jnp.maximum(margin - dist, 0.0)
    hinge = clamped * clamped

    if mask_tail:
        row_ids = lax.broadcasted_iota(jnp.int32, dist_sq.shape, 0) + i * block_rows
        valid = row_ids < n_rows
        dist_sq = jnp.where(valid, dist_sq, 0.0)
        hinge = jnp.where(valid, hinge, 0.0)

    y_row = y_ref[...].astype(jnp.float32).reshape(1, block_rows)
    weighted = jnp.dot(
        y_row, dist_sq - hinge, preferred_element_type=jnp.float32
    )
    partial = weighted + jnp.sum(hinge, keepdims=True)

    out_ref[...] = jnp.broadcast_to(partial, out_ref.shape)


def _round_up(v, m):
    return ((v + m - 1) // m) * m


def kernel(x0, x1, y):
    n, d = x0.shape
    margin = 1.0

    tn = min(_BLOCK_ROWS, _round_up(n, _SUBLANES))
    grid_n = pl.cdiv(n, tn)
    mask_tail = (grid_n * tn) != n

    y_flat = y.reshape(-1)
    if grid_n * tn != n:
        y_flat = jnp.pad(y_flat.astype(jnp.float32), (0, grid_n * tn - n))

    body = functools.partial(
        _loss_block_kernel,
        margin=float(margin),
        n_rows=int(n),
        block_rows=int(tn),
        mask_tail=bool(mask_tail),
    )

    partials = pl.pallas_call(
        body,
        out_shape=jax.ShapeDtypeStruct((grid_n * _SUBLANES, _LANES), jnp.float32),
        grid=(grid_n,),
        in_specs=[
            pl.BlockSpec((tn, d), lambda i: (i, 0)),
            pl.BlockSpec((tn, d), lambda i: (i, 0)),
            pl.BlockSpec((tn,), lambda i: (i,)),
        ],
        out_specs=pl.BlockSpec((_SUBLANES, _LANES), lambda i: (i, 0)),
        compiler_params=pltpu.CompilerParams(
            dimension_semantics=("parallel",),
        ),
    )(x0, x1, y_flat)

    per_block = partials.reshape(grid_n, _SUBLANES, _LANES)[:, 0, 0]
    return jnp.sum(per_block) * (0.5 / n)


# speedup vs baseline: 1.6261x; 1.0304x over previous
import functools

import jax
import jax.numpy as jnp
from jax import lax
from jax.experimental import pallas as pl
from jax.experimental.pallas import tpu as pltpu

_LANES = 128
_SUBLANES = 8
_BLOCK_ROWS = 8192


def _loss_block_kernel(
    x0_ref, x1_ref, y_ref, out_ref, *, margin, n_rows, block_rows, mask_tail
):
    i = pl.program_id(0)

    x0 = x0_ref[...].astype(jnp.float32)
    x1 = x1_ref[...].astype(jnp.float32)
    diff = x0 - x1
    sq = diff * diff

    ones = jnp.ones((sq.shape[1], 1), jnp.float32)
    dist_sq = jnp.dot(sq, ones, preferred_element_type=jnp.float32)
    dist = jnp.sqrt(dist_sq)
    clamped = jnp.maximum(margin - dist, 0.0)
    hinge = clamped * clamped

    if mask_tail:
        row_ids = lax.broadcasted_iota(jnp.int32, dist_sq.shape, 0) + i * block_rows
        valid = row_ids < n_rows
        dist_sq = jnp.where(valid, dist_sq, 0.0)
        hinge = jnp.where(valid, hinge, 0.0)

    y_row = y_ref[...].astype(jnp.float32).reshape(1, block_rows)
    weighted = jnp.dot(
        y_row, dist_sq - hinge, preferred_element_type=jnp.float32
    )
    partial = weighted + jnp.sum(hinge, keepdims=True)

    @pl.when(i == 0)
    def _():
        out_ref[...] = jnp.zeros_like(out_ref)

    out_ref[...] += jnp.broadcast_to(partial, out_ref.shape)

    @pl.when(i == pl.num_programs(0) - 1)
    def _():
        out_ref[...] *= 0.5 / n_rows


def _round_up(v, m):
    return ((v + m - 1) // m) * m


def kernel(x0, x1, y):
    n, d = x0.shape
    margin = 1.0

    tn = min(_BLOCK_ROWS, _round_up(n, _SUBLANES))
    grid_n = pl.cdiv(n, tn)
    mask_tail = (grid_n * tn) != n

    y_flat = y.reshape(-1)
    if grid_n * tn != n:
        y_flat = jnp.pad(y_flat.astype(jnp.float32), (0, grid_n * tn - n))

    body = functools.partial(
        _loss_block_kernel,
        margin=float(margin),
        n_rows=int(n),
        block_rows=int(tn),
        mask_tail=bool(mask_tail),
    )

    acc = pl.pallas_call(
        body,
        out_shape=jax.ShapeDtypeStruct((_SUBLANES, _LANES), jnp.float32),
        grid=(grid_n,),
        in_specs=[
            pl.BlockSpec((tn, d), lambda i: (i, 0)),
            pl.BlockSpec((tn, d), lambda i: (i, 0)),
            pl.BlockSpec((tn,), lambda i: (i,)),
        ],
        out_specs=pl.BlockSpec((_SUBLANES, _LANES), lambda i: (0, 0)),
        compiler_params=pltpu.CompilerParams(
            dimension_semantics=("arbitrary",),
        ),
    )(x0, x1, y_flat)

    return acc[0, 0]


# speedup vs baseline: 1.6724x; 1.0285x over previous
import functools

import jax
import jax.numpy as jnp
from jax import lax
from jax.experimental import pallas as pl
from jax.experimental.pallas import tpu as pltpu

_LANES = 128
_SUBLANES = 8
_BLOCK_ROWS = 8192


def _loss_block_kernel(
    x0_ref, x1_ref, y_ref, out_ref, *, margin, n_rows, block_rows, mask_tail
):
    i = pl.program_id(0)

    x0 = x0_ref[...].astype(jnp.float32)
    x1 = x1_ref[...].astype(jnp.float32)
    diff = x0 - x1
    sq = diff * diff

    ones = jnp.ones((sq.shape[1], 1), jnp.float32)
    dist_sq = jnp.dot(sq, ones, preferred_element_type=jnp.float32)
    dist = jnp.sqrt(dist_sq)
    clamped = jnp.maximum(margin - dist, 0.0)
    hinge = clamped * clamped

    if mask_tail:
        row_ids = lax.broadcasted_iota(jnp.int32, dist_sq.shape, 0) + i * block_rows
        valid = row_ids < n_rows
        dist_sq = jnp.where(valid, dist_sq, 0.0)
        hinge = jnp.where(valid, hinge, 0.0)

    y_row = y_ref[...].astype(jnp.float32).reshape(1, block_rows)
    weighted = jnp.dot(
        y_row, dist_sq - hinge, preferred_element_type=jnp.float32
    )
    partial = weighted + jnp.sum(hinge, keepdims=True)

    @pl.when(i == 0)
    def _():
        out_ref[...] = jnp.zeros_like(out_ref)

    out_ref[...] += partial

    @pl.when(i == pl.num_programs(0) - 1)
    def _():
        out_ref[...] *= 0.5 / n_rows


def _round_up(v, m):
    return ((v + m - 1) // m) * m


def kernel(x0, x1, y):
    n, d = x0.shape
    margin = 1.0

    tn = min(_BLOCK_ROWS, _round_up(n, _SUBLANES))
    grid_n = pl.cdiv(n, tn)
    mask_tail = (grid_n * tn) != n

    y_flat = y.reshape(-1)
    if grid_n * tn != n:
        y_flat = jnp.pad(y_flat.astype(jnp.float32), (0, grid_n * tn - n))

    body = functools.partial(
        _loss_block_kernel,
        margin=float(margin),
        n_rows=int(n),
        block_rows=int(tn),
        mask_tail=bool(mask_tail),
    )

    acc = pl.pallas_call(
        body,
        out_shape=jax.ShapeDtypeStruct((1, 1), jnp.float32),
        grid=(grid_n,),
        in_specs=[
            pl.BlockSpec((tn, d), lambda i: (i, 0)),
            pl.BlockSpec((tn, d), lambda i: (i, 0)),
            pl.BlockSpec((tn,), lambda i: (i,)),
        ],
        out_specs=pl.BlockSpec((1, 1), lambda i: (0, 0)),
        compiler_params=pltpu.CompilerParams(
            dimension_semantics=("arbitrary",),
        ),
    )(x0, x1, y_flat)

    return acc.reshape(())


# speedup vs baseline: 1.7081x; 1.0213x over previous
import functools

import jax
import jax.numpy as jnp
from jax import lax
from jax.experimental import pallas as pl
from jax.experimental.pallas import tpu as pltpu

_LANES = 128
_SUBLANES = 8
_BLOCK_ROWS = 4096


def _loss_block_kernel(
    x0_ref, x1_ref, y_ref, out_ref, *, margin, n_rows, block_rows, mask_tail
):
    i = pl.program_id(0)

    x0 = x0_ref[...].astype(jnp.float32)
    x1 = x1_ref[...].astype(jnp.float32)
    diff = x0 - x1
    sq = diff * diff

    ones = jnp.ones((sq.shape[1], 1), jnp.float32)
    dist_sq = jnp.dot(sq, ones, preferred_element_type=jnp.float32)
    dist = jnp.sqrt(dist_sq)
    clamped = jnp.maximum(margin - dist, 0.0)
    hinge = clamped * clamped

    if mask_tail:
        row_ids = lax.broadcasted_iota(jnp.int32, dist_sq.shape, 0) + i * block_rows
        valid = row_ids < n_rows
        dist_sq = jnp.where(valid, dist_sq, 0.0)
        hinge = jnp.where(valid, hinge, 0.0)

    y_row = y_ref[...].astype(jnp.float32).reshape(1, block_rows)
    weighted = jnp.dot(
        y_row, dist_sq - hinge, preferred_element_type=jnp.float32
    )
    partial = weighted + jnp.sum(hinge, keepdims=True)

    @pl.when(i == 0)
    def _():
        out_ref[...] = jnp.zeros_like(out_ref)

    out_ref[...] += partial

    @pl.when(i == pl.num_programs(0) - 1)
    def _():
        out_ref[...] *= 0.5 / n_rows


def _round_up(v, m):
    return ((v + m - 1) // m) * m


def kernel(x0, x1, y):
    n, d = x0.shape
    margin = 1.0

    tn = min(_BLOCK_ROWS, _round_up(n, _SUBLANES))
    grid_n = pl.cdiv(n, tn)
    mask_tail = (grid_n * tn) != n

    y_flat = y.reshape(-1)
    if grid_n * tn != n:
        y_flat = jnp.pad(y_flat.astype(jnp.float32), (0, grid_n * tn - n))

    body = functools.partial(
        _loss_block_kernel,
        margin=float(margin),
        n_rows=int(n),
        block_rows=int(tn),
        mask_tail=bool(mask_tail),
    )

    acc = pl.pallas_call(
        body,
        out_shape=jax.ShapeDtypeStruct((1, 1), jnp.float32),
        grid=(grid_n,),
        in_specs=[
            pl.BlockSpec((tn, d), lambda i: (i, 0)),
            pl.BlockSpec((tn, d), lambda i: (i, 0)),
            pl.BlockSpec((tn,), lambda i: (i,)),
        ],
        out_specs=pl.BlockSpec((1, 1), lambda i: (0, 0)),
        compiler_params=pltpu.CompilerParams(
            dimension_semantics=("arbitrary",),
        ),
    )(x0, x1, y_flat)

    return acc.reshape(())
